# Initial kernel scaffold; baseline (speedup 1.0000x reference)
import functools

import jax
import jax.numpy as jnp
from jax import lax
from jax.experimental import pallas as pl
from jax.experimental.pallas import tpu as pltpu
from jax.experimental.pallas import tpu_sc as plsc

N_NODES = 10000
D_IN = 128
D_OUT = 128

N_PAD = 10016
D_PAD = 144
NC = 2
NS = 16
NW = NC * NS
CHUNK = 128
CHUNKS_PER_W = 80
EDGES_PER_W = CHUNK * CHUNKS_PER_W
E_PAD = EDGES_PER_W * NW
ROWS_PER_TILE = N_PAD // NS


def _sc_accumulate(x_aug, src, dst):
    mesh = plsc.VectorSubcoreMesh(core_axis_name="c", subcore_axis_name="s")

    @functools.partial(
        pl.kernel,
        mesh=mesh,
        out_type=jax.ShapeDtypeStruct((NC, N_PAD, D_PAD), jnp.float32),
        scratch_types=[
            pltpu.VMEM_SHARED((N_PAD, D_PAD), jnp.float32),
            pltpu.VMEM((2, CHUNK), jnp.int32),
            pltpu.VMEM((2, CHUNK), jnp.int32),
            pltpu.VMEM((2, CHUNK, D_PAD), jnp.float32),
            pltpu.VMEM((CHUNK, D_PAD), jnp.float32),
            pltpu.SemaphoreType.DMA,
            pltpu.SemaphoreType.DMA,
        ],
    )
    def k(x_hbm, src_hbm, dst_hbm, out_hbm, acc, sidx, didx, rows, zbuf, sem0, sem1):
        c = lax.axis_index("c")
        s = lax.axis_index("s")
        wid = s * NC + c
        ebase = wid * EDGES_PER_W
        sems = (sem0, sem1)

        zv = jnp.zeros((16,), jnp.float32)

        def zfill(i, carry):
            for j in range(D_PAD // 16):
                zbuf[i, pl.ds(j * 16, 16)] = zv
            return carry

        lax.fori_loop(0, CHUNK, zfill, 0)
        rbase = s * ROWS_PER_TILE
        nfull = ROWS_PER_TILE // CHUNK
        rem = ROWS_PER_TILE - nfull * CHUNK
        for t in range(nfull):
            pltpu.sync_copy(zbuf, acc.at[pl.ds(rbase + t * CHUNK, CHUNK)])
        pltpu.sync_copy(zbuf.at[pl.ds(0, rem)],
                        acc.at[pl.ds(rbase + nfull * CHUNK, rem)])
        plsc.subcore_barrier()

        def start(i, b):
            off = ebase + i * CHUNK
            pltpu.sync_copy(src_hbm.at[pl.ds(off, CHUNK)], sidx.at[b])
            pltpu.sync_copy(dst_hbm.at[pl.ds(off, CHUNK)], didx.at[b])
            pltpu.async_copy(x_hbm.at[sidx.at[b]], rows.at[b], sems[b])

        start(0, 0)

        def step(t, carry):
            g = t * 2
            for b in range(2):
                i = g + b
                pltpu.make_async_copy(x_hbm.at[sidx.at[b]], rows.at[b],
                                      sems[b]).wait()

                @pl.when(i + 1 < CHUNKS_PER_W)
                def _():
                    off = ebase + (i + 1) * CHUNK
                    nb = 1 - b
                    pltpu.sync_copy(src_hbm.at[pl.ds(off, CHUNK)], sidx.at[nb])
                    pltpu.sync_copy(dst_hbm.at[pl.ds(off, CHUNK)], didx.at[nb])
                    pltpu.async_copy(x_hbm.at[sidx.at[nb]], rows.at[nb], sems[nb])

                pltpu.sync_copy(rows.at[b], acc.at[didx.at[b]], add=True)
            return carry

        lax.fori_loop(0, CHUNKS_PER_W // 2, step, 0)
        plsc.subcore_barrier()

        pltpu.sync_copy(acc.at[pl.ds(rbase, ROWS_PER_TILE)],
                        out_hbm.at[c, pl.ds(rbase, ROWS_PER_TILE)])

    return k(x_aug, src, dst)


def _tc_finish(partial, W, b2):
    BR = 500
    grid = (N_NODES // BR,)

    def body(p_ref, w_ref, b_ref, o_ref):
        p = p_ref[0] + p_ref[1]
        feat = p[:, :D_IN]
        deg = p[:, D_IN:D_IN + 1]
        h = feat / jnp.maximum(deg, 1.0)
        o_ref[...] = (
            jnp.dot(h, w_ref[...], preferred_element_type=jnp.float32) + b_ref[...]
        )

    return pl.pallas_call(
        body,
        grid=grid,
        in_specs=[
            pl.BlockSpec((NC, BR, D_PAD), lambda i: (0, i, 0)),
            pl.BlockSpec((D_IN, D_OUT), lambda i: (0, 0)),
            pl.BlockSpec((1, D_OUT), lambda i: (0, 0)),
        ],
        out_specs=pl.BlockSpec((BR, D_OUT), lambda i: (i, 0)),
        out_shape=jax.ShapeDtypeStruct((N_NODES, D_OUT), jnp.float32),
    )(partial, W, b2)


def kernel(x, edge_index, W, b):
    n_extra = E_PAD - edge_index.shape[1]
    pad_idx = jnp.full((n_extra,), N_NODES, dtype=jnp.int32)
    src = jnp.concatenate([edge_index[0], pad_idx])
    dst = jnp.concatenate([edge_index[1], pad_idx])
    x_aug = jnp.zeros((N_PAD, D_PAD), jnp.float32)
    x_aug = x_aug.at[:N_NODES, :D_IN].set(x)
    x_aug = x_aug.at[:N_NODES, D_IN].set(1.0)
    partial = _sc_accumulate(x_aug, src, dst)
    return _tc_finish(partial, W, b.reshape(1, D_OUT))


# speedup vs baseline: 3.5395x; 3.5395x over previous
import functools

import jax
import jax.numpy as jnp
from jax import lax
from jax.experimental import pallas as pl
from jax.experimental.pallas import tpu as pltpu
from jax.experimental.pallas import tpu_sc as plsc

N_NODES = 10000
D_IN = 128
D_OUT = 128

N_PAD = 10112
D_PAD = 144
NC = 2
NS = 16
NW = NC * NS
CHUNK = 128
CHUNKS_PER_W = 80
EDGES_PER_W = CHUNK * CHUNKS_PER_W
E_PAD = EDGES_PER_W * NW
ROWS_PER_TILE = N_PAD // NS


def _sc_accumulate(x_aug, src, dst):
    mesh = plsc.VectorSubcoreMesh(core_axis_name="c", subcore_axis_name="s")

    @functools.partial(
        pl.kernel,
        mesh=mesh,
        compiler_params=pltpu.CompilerParams(use_tc_tiling_on_sc=False),
        out_type=jax.ShapeDtypeStruct((NC, N_PAD, D_PAD), jnp.float32),
        scratch_types=[
            pltpu.VMEM_SHARED((N_PAD, D_PAD), jnp.float32),
            pltpu.VMEM((2, CHUNK), jnp.int32),
            pltpu.VMEM((2, CHUNK), jnp.int32),
            pltpu.VMEM((2, CHUNK, D_PAD), jnp.float32),
            pltpu.SemaphoreType.DMA,
            pltpu.SemaphoreType.DMA,
        ],
    )
    def k(x_hbm, src_hbm, dst_hbm, out_hbm, acc, sidx, didx, rows, sem0, sem1):
        c = lax.axis_index("c")
        s = lax.axis_index("s")
        wid = s * NC + c
        ebase = wid * EDGES_PER_W
        sems = (sem0, sem1)

        zv = jnp.zeros((16,), jnp.float32)

        def zfill(i, carry):
            for j in range(D_PAD // 16):
                rows[0, i, pl.ds(j * 16, 16)] = zv
            return carry

        lax.fori_loop(0, CHUNK, zfill, 0)
        rbase = s * ROWS_PER_TILE
        for t in range(4):
            pltpu.sync_copy(rows.at[0], acc.at[pl.ds(rbase + t * CHUNK, CHUNK)])
        pltpu.sync_copy(rows.at[0],
                        acc.at[pl.ds(rbase + ROWS_PER_TILE - CHUNK, CHUNK)])
        plsc.subcore_barrier()

        def start(i, b):
            off = ebase + i * CHUNK
            pltpu.sync_copy(src_hbm.at[pl.ds(off, CHUNK)], sidx.at[b])
            pltpu.sync_copy(dst_hbm.at[pl.ds(off, CHUNK)], didx.at[b])
            pltpu.async_copy(x_hbm.at[sidx.at[b]], rows.at[b], sems[b])

        start(0, 0)

        def step(t, carry):
            g = t * 2
            for b in range(2):
                i = g + b
                pltpu.make_async_copy(x_hbm.at[sidx.at[b]], rows.at[b],
                                      sems[b]).wait()

                @pl.when(i + 1 < CHUNKS_PER_W)
                def _():
                    off = ebase + (i + 1) * CHUNK
                    nb = 1 - b
                    pltpu.sync_copy(src_hbm.at[pl.ds(off, CHUNK)], sidx.at[nb])
                    pltpu.sync_copy(dst_hbm.at[pl.ds(off, CHUNK)], didx.at[nb])
                    pltpu.async_copy(x_hbm.at[sidx.at[nb]], rows.at[nb], sems[nb])

                pltpu.sync_copy(rows.at[b], acc.at[didx.at[b]], add=True)
            return carry

        lax.fori_loop(0, CHUNKS_PER_W // 2, step, 0)
        plsc.subcore_barrier()

        pltpu.sync_copy(acc.at[pl.ds(rbase, ROWS_PER_TILE)],
                        out_hbm.at[c, pl.ds(rbase, ROWS_PER_TILE)])

    return k(x_aug, src, dst)


def _tc_finish(partial, W, b2):
    BR = 1000
    grid = (N_NODES // BR,)

    def body(p_ref, w_ref, b_ref, o_ref):
        p = p_ref[0] + p_ref[1]
        feat = p[:, :D_IN]
        deg = p[:, D_IN:D_IN + 1]
        h = feat / jnp.maximum(deg, 1.0)
        o_ref[...] = (
            jnp.dot(h, w_ref[...], preferred_element_type=jnp.float32) + b_ref[...]
        )

    return pl.pallas_call(
        body,
        grid=grid,
        in_specs=[
            pl.BlockSpec((NC, BR, D_PAD), lambda i: (0, i, 0)),
            pl.BlockSpec((D_IN, D_OUT), lambda i: (0, 0)),
            pl.BlockSpec((1, D_OUT), lambda i: (0, 0)),
        ],
        out_specs=pl.BlockSpec((BR, D_OUT), lambda i: (i, 0)),
        out_shape=jax.ShapeDtypeStruct((N_NODES, D_OUT), jnp.float32),
    )(partial, W, b2)


def kernel(x, edge_index, W, b):
    n_extra = E_PAD - edge_index.shape[1]
    pad_idx = jnp.full((n_extra,), N_NODES, dtype=jnp.int32)
    src = jnp.concatenate([edge_index[0], pad_idx])
    dst = jnp.concatenate([edge_index[1], pad_idx])
    x_aug = jnp.zeros((N_PAD, D_PAD), jnp.float32)
    x_aug = x_aug.at[:N_NODES, :D_IN].set(x)
    x_aug = x_aug.at[:N_NODES, D_IN].set(1.0)
    partial = _sc_accumulate(x_aug, src, dst)
    return _tc_finish(partial, W, b.reshape(1, D_OUT))


# speedup vs baseline: 3.5844x; 1.0127x over previous
import functools

import jax
import jax.numpy as jnp
from jax import lax
from jax.experimental import pallas as pl
from jax.experimental.pallas import tpu as pltpu
from jax.experimental.pallas import tpu_sc as plsc

N_NODES = 10000
D_IN = 128
D_OUT = 128

N_PAD = 10112
DEG_W = 8
NC = 2
NS = 16
NW = NC * NS
CHUNK = 128
CHUNKS_PER_W = 80
EDGES_PER_W = CHUNK * CHUNKS_PER_W
E_PAD = EDGES_PER_W * NW
ROWS_PER_TILE = N_PAD // NS


def _sc_accumulate(x, src, dst, zsum, zdeg, ones8):
    mesh = plsc.VectorSubcoreMesh(core_axis_name="c", subcore_axis_name="s")

    @functools.partial(
        pl.kernel,
        mesh=mesh,
        compiler_params=pltpu.CompilerParams(use_tc_tiling_on_sc=False),
        out_type=(
            jax.ShapeDtypeStruct((NC, N_PAD, D_IN), jnp.float32),
            jax.ShapeDtypeStruct((NC, N_PAD, DEG_W), jnp.float32),
        ),
        scratch_types=[
            pltpu.VMEM_SHARED((N_PAD, D_IN), jnp.float32),
            pltpu.VMEM_SHARED((N_PAD, DEG_W), jnp.float32),
            pltpu.VMEM((2, CHUNK), jnp.int32),
            pltpu.VMEM((2, CHUNK), jnp.int32),
            pltpu.VMEM((2, CHUNK, D_IN), jnp.float32),
            pltpu.VMEM((CHUNK, DEG_W), jnp.float32),
            pltpu.SemaphoreType.DMA,
            pltpu.SemaphoreType.DMA,
        ],
    )
    def k(x_hbm, src_hbm, dst_hbm, zsum_hbm, zdeg_hbm, ones_hbm,
          sum_out, deg_out, acc, dacc, sidx, didx, rows, ones_v, sem0, sem1):
        c = lax.axis_index("c")
        s = lax.axis_index("s")
        wid = s * NC + c
        ebase = wid * EDGES_PER_W
        sems = (sem0, sem1)

        rbase = s * ROWS_PER_TILE
        rs = pl.ds(rbase, ROWS_PER_TILE)
        pltpu.sync_copy(zsum_hbm.at[rs], acc.at[rs])
        pltpu.sync_copy(zdeg_hbm.at[rs], dacc.at[rs])
        pltpu.sync_copy(ones_hbm, ones_v)
        plsc.subcore_barrier()

        def start(i, b):
            off = ebase + i * CHUNK
            pltpu.sync_copy(src_hbm.at[pl.ds(off, CHUNK)], sidx.at[b])
            pltpu.sync_copy(dst_hbm.at[pl.ds(off, CHUNK)], didx.at[b])
            pltpu.async_copy(x_hbm.at[sidx.at[b]], rows.at[b], sems[b])

        start(0, 0)

        def step(t, carry):
            g = t * 2
            for b in range(2):
                i = g + b
                pltpu.make_async_copy(x_hbm.at[sidx.at[b]], rows.at[b],
                                      sems[b]).wait()

                @pl.when(i + 1 < CHUNKS_PER_W)
                def _():
                    start(i + 1, 1 - b)

                pltpu.sync_copy(rows.at[b], acc.at[didx.at[b]], add=True)
                pltpu.sync_copy(ones_v, dacc.at[didx.at[b]], add=True)
            return carry

        lax.fori_loop(0, CHUNKS_PER_W // 2, step, 0)
        plsc.subcore_barrier()

        pltpu.sync_copy(acc.at[rs], sum_out.at[c, rs])
        pltpu.sync_copy(dacc.at[rs], deg_out.at[c, rs])

    return k(x, src, dst, zsum, zdeg, ones8)


def _tc_finish(psum, pdeg, W, b2):
    BR = 1000
    grid = (N_NODES // BR,)

    def body(p_ref, d_ref, w_ref, b_ref, o_ref):
        p = p_ref[0] + p_ref[1]
        d = d_ref[0] + d_ref[1]
        deg = d[:, 0:1]
        h = p / jnp.maximum(deg, 1.0)
        o_ref[...] = (
            jnp.dot(h, w_ref[...], preferred_element_type=jnp.float32) + b_ref[...]
        )

    return pl.pallas_call(
        body,
        grid=grid,
        in_specs=[
            pl.BlockSpec((NC, BR, D_IN), lambda i: (0, i, 0)),
            pl.BlockSpec((NC, BR, DEG_W), lambda i: (0, i, 0)),
            pl.BlockSpec((D_IN, D_OUT), lambda i: (0, 0)),
            pl.BlockSpec((1, D_OUT), lambda i: (0, 0)),
        ],
        out_specs=pl.BlockSpec((BR, D_OUT), lambda i: (i, 0)),
        out_shape=jax.ShapeDtypeStruct((N_NODES, D_OUT), jnp.float32),
    )(psum, pdeg, W, b2)


def kernel(x, edge_index, W, b):
    n_extra = E_PAD - edge_index.shape[1]
    src = jnp.concatenate([edge_index[0], jnp.zeros((n_extra,), jnp.int32)])
    dst = jnp.concatenate([edge_index[1],
                           jnp.full((n_extra,), N_NODES, dtype=jnp.int32)])
    zsum = jnp.zeros((N_PAD, D_IN), jnp.float32)
    zdeg = jnp.zeros((N_PAD, DEG_W), jnp.float32)
    ones8 = jnp.ones((CHUNK, DEG_W), jnp.float32)
    psum, pdeg = _sc_accumulate(x, src, dst, zsum, zdeg, ones8)
    return _tc_finish(psum, pdeg, W, b.reshape(1, D_OUT))
